# baseline (speedup 1.0000x reference)
from functools import partial

import jax
import jax.numpy as jnp
from jax.experimental import pallas as pl
from jax.experimental.pallas import tpu as pltpu


def _lstm_body(x_ref, wih_ref, whh_ref, b_ref, out_ref, h_ref, c_ref,
               gx_ref, *, hidden, tc, bb):
    h4 = 4 * hidden

    @pl.when(pl.program_id(0) == 0)
    def _init():
        h_ref[...] = jnp.zeros_like(h_ref)
        c_ref[...] = jnp.zeros_like(c_ref)

    xs = x_ref[...].reshape(tc * bb, x_ref.shape[-1])
    gx_ref[...] = (
        jnp.dot(xs, wih_ref[...], preferred_element_type=jnp.float32)
        + b_ref[...]
    )

    hb = bb // 2
    lane = jax.lax.broadcasted_iota(jnp.int32, (hb, h4), 1)
    is_g = (lane // hidden) == 2
    a_scale = jnp.where(is_g, 1.0, 0.5).astype(jnp.float32)
    a_shift = jnp.where(is_g, 0.0, 0.5).astype(jnp.float32)

    def _half(row, h, c):
        pre = gx_ref[pl.ds(row, hb), :] + jnp.dot(
            h.astype(whh_ref.dtype), whh_ref[...],
            preferred_element_type=jnp.float32)
        act = jnp.tanh(pre) * a_scale + a_shift
        gi = act[:, 0:hidden]
        gf = act[:, hidden:2 * hidden]
        gg = act[:, 2 * hidden:3 * hidden]
        go = act[:, 3 * hidden:4 * hidden]
        c_new = gf * c + gi * gg
        h_new = go * jnp.tanh(c_new)
        return h_new, c_new

    def _step(t, carry):
        h1, c1, h2, c2 = carry
        h1n, c1n = _half(t * bb, h1, c1)
        h2n, c2n = _half(t * bb + hb, h2, c2)
        out_ref[t, 0:hb] = h1n
        out_ref[t, hb:bb] = h2n
        return h1n, c1n, h2n, c2n

    carry0 = (h_ref[0:hb], c_ref[0:hb], h_ref[hb:bb], c_ref[hb:bb])
    h1f, c1f, h2f, c2f = jax.lax.fori_loop(
        0, tc, _step, carry0, unroll=True)
    h_ref[0:hb] = h1f
    c_ref[0:hb] = c1f
    h_ref[hb:bb] = h2f
    c_ref[hb:bb] = c2f


def kernel(x, w_ih, w_hh, bias):
    T, B, I = x.shape
    H = w_hh.shape[0]
    G = 4 * H
    Bb = B
    Tc = 16
    assert T % Tc == 0

    gate_scale = jnp.concatenate([
        jnp.full((1, H), s, jnp.float32)
        for s in (0.5, 0.5, 1.0, 0.5)], axis=1)
    wih = (w_ih * gate_scale).astype(jnp.bfloat16)
    whh = (w_hh * gate_scale).astype(jnp.bfloat16)
    b = (bias * gate_scale).astype(jnp.float32)
    xb = x.astype(jnp.bfloat16)

    vmem_limit = int(min(
        2 * (2 * Tc * Bb * I * 2
             + I * G * 2 + H * G * 2
             + 2 * Tc * Bb * H * 4
             + Tc * Bb * G * 4
             + 2 * Bb * H * 4),
        100 * 2**20))

    out = pl.pallas_call(
        partial(_lstm_body, hidden=H, tc=Tc, bb=Bb),
        out_shape=jax.ShapeDtypeStruct((T, B, H), jnp.float32),
        grid=(T // Tc,),
        in_specs=[
            pl.BlockSpec((Tc, Bb, I), lambda t: (t, 0, 0)),
            pl.BlockSpec((I, G), lambda t: (0, 0)),
            pl.BlockSpec((H, G), lambda t: (0, 0)),
            pl.BlockSpec((1, G), lambda t: (0, 0)),
        ],
        out_specs=pl.BlockSpec((Tc, Bb, H), lambda t: (t, 0, 0)),
        scratch_shapes=[
            pltpu.VMEM((Bb, H), jnp.float32),
            pltpu.VMEM((Bb, H), jnp.float32),
            pltpu.VMEM((Tc * Bb, G), jnp.float32),
        ],
        compiler_params=pltpu.CompilerParams(
            dimension_semantics=(
                pltpu.GridDimensionSemantics.ARBITRARY,
            ),
            vmem_limit_bytes=vmem_limit,
        ),
    )(xb, wih, whh, b)
    return out


# speedup vs baseline: 1.0819x; 1.0819x over previous
from functools import partial

import jax
import jax.numpy as jnp
from jax.experimental import pallas as pl
from jax.experimental.pallas import tpu as pltpu


def _lstm_body(x_ref, wih_ref, whh_ref, b_ref, out_ref, h_ref, c_ref,
               gx_ref, *, hidden, tc, bb):
    h4 = 4 * hidden

    @pl.when(pl.program_id(0) == 0)
    def _init():
        h_ref[...] = jnp.zeros_like(h_ref)
        c_ref[...] = jnp.zeros_like(c_ref)

    xs = x_ref[...].reshape(tc * bb, x_ref.shape[-1])
    gx_ref[...] = (
        jnp.dot(xs, wih_ref[...], preferred_element_type=jnp.float32)
        + b_ref[...]
    )

    lane = jax.lax.broadcasted_iota(jnp.int32, (bb, h4), 1)
    is_g = (lane // hidden) == 2
    a_scale = jnp.where(is_g, 1.0, 0.5).astype(jnp.float32)
    a_shift = jnp.where(is_g, 0.0, 0.5).astype(jnp.float32)

    def _step(t, carry):
        h, c = carry
        pre = gx_ref[pl.ds(t * bb, bb), :] + jnp.dot(
            h.astype(whh_ref.dtype), whh_ref[...],
            preferred_element_type=jnp.float32)
        act = jnp.tanh(pre) * a_scale + a_shift
        gi = act[:, 0:hidden]
        gf = act[:, hidden:2 * hidden]
        gg = act[:, 2 * hidden:3 * hidden]
        go = act[:, 3 * hidden:4 * hidden]
        c_new = gf * c + gi * gg
        h_new = go * jnp.tanh(c_new)
        out_ref[t] = h_new
        return h_new, c_new

    h_fin, c_fin = jax.lax.fori_loop(
        0, tc, _step, (h_ref[...], c_ref[...]), unroll=True)
    h_ref[...] = h_fin
    c_ref[...] = c_fin


def kernel(x, w_ih, w_hh, bias):
    T, B, I = x.shape
    H = w_hh.shape[0]
    G = 4 * H
    Bb = B
    Tc = 16
    assert T % Tc == 0

    gate_scale = jnp.concatenate([
        jnp.full((1, H), s, jnp.float32)
        for s in (0.5, 0.5, 1.0, 0.5)], axis=1)
    wih = (w_ih * gate_scale).astype(jnp.bfloat16)
    whh = (w_hh * gate_scale).astype(jnp.bfloat16)
    b = (bias * gate_scale).astype(jnp.float32)
    xb = x.astype(jnp.bfloat16)

    vmem_limit = int(min(
        2 * (2 * Tc * Bb * I * 2
             + I * G * 2 + H * G * 2
             + 2 * Tc * Bb * H * 4
             + Tc * Bb * G * 4
             + 2 * Bb * H * 4),
        100 * 2**20))

    out = pl.pallas_call(
        partial(_lstm_body, hidden=H, tc=Tc, bb=Bb),
        out_shape=jax.ShapeDtypeStruct((T, B, H), jnp.float32),
        grid=(T // Tc,),
        in_specs=[
            pl.BlockSpec((Tc, Bb, I), lambda t: (t, 0, 0)),
            pl.BlockSpec((I, G), lambda t: (0, 0)),
            pl.BlockSpec((H, G), lambda t: (0, 0)),
            pl.BlockSpec((1, G), lambda t: (0, 0)),
        ],
        out_specs=pl.BlockSpec((Tc, Bb, H), lambda t: (t, 0, 0)),
        scratch_shapes=[
            pltpu.VMEM((Bb, H), jnp.float32),
            pltpu.VMEM((Bb, H), jnp.float32),
            pltpu.VMEM((Tc * Bb, G), jnp.float32),
        ],
        compiler_params=pltpu.CompilerParams(
            dimension_semantics=(
                pltpu.GridDimensionSemantics.ARBITRARY,
            ),
            vmem_limit_bytes=vmem_limit,
        ),
    )(xb, wih, whh, b)
    return out
